# baseline (speedup 1.0000x reference)
import functools

import jax
import jax.numpy as jnp
from jax import lax
from jax.experimental import pallas as pl
from jax.experimental.pallas import tpu as pltpu
from jax.experimental.pallas import tpu_sc as plsc


_NCHUNK = 4


@functools.cache
def _make_gather(V, D, B):
    info = plsc.get_sparse_core_info()
    NC, NS = info.num_cores, info.num_subcores
    NW = NC * NS
    assert B % NW == 0
    b_per_w = B // NW
    nch = _NCHUNK
    rc = b_per_w // nch
    assert rc * nch == b_per_w and rc % 8 == 0
    mesh = plsc.VectorSubcoreMesh(core_axis_name="c", subcore_axis_name="s")

    @functools.partial(
        pl.kernel,
        mesh=mesh,
        out_type=jax.ShapeDtypeStruct((B, D), jnp.float32),
        scratch_types=[
            pltpu.VMEM((b_per_w,), jnp.int32),
            pltpu.VMEM((b_per_w, D), jnp.float32),
        ]
        + [pltpu.SemaphoreType.DMA] * (2 * nch),
    )
    def k(table_hbm, idx_hbm, out_hbm, idx_v, rows_v, *sems):
        gsem, osem = sems[:nch], sems[nch:]
        wid = lax.axis_index("s") * NC + lax.axis_index("c")
        base = wid * b_per_w
        pltpu.sync_copy(idx_hbm.at[pl.ds(base, b_per_w)], idx_v)
        gathers = [
            pltpu.async_copy(
                table_hbm.at[idx_v.at[pl.ds(c * rc, rc)]],
                rows_v.at[pl.ds(c * rc, rc)],
                gsem[c],
            )
            for c in range(nch)
        ]
        outs = []
        for c in range(nch):
            gathers[c].wait()
            outs.append(
                pltpu.async_copy(
                    rows_v.at[pl.ds(c * rc, rc)],
                    out_hbm.at[pl.ds(base + c * rc, rc)],
                    osem[c],
                )
            )
        for o in outs:
            o.wait()

    return k


def kernel(labels, table):
    V, D = table.shape
    (B,) = labels.shape
    k = _make_gather(V, D, B)
    return k(table, labels.astype(jnp.int32))


# speedup vs baseline: 1.0078x; 1.0078x over previous
import functools

import jax
import jax.numpy as jnp
from jax import lax
from jax.experimental import pallas as pl
from jax.experimental.pallas import tpu as pltpu
from jax.experimental.pallas import tpu_sc as plsc


_NCHUNK = 4


@functools.cache
def _make_gather(V, D, B):
    info = plsc.get_sparse_core_info()
    NC, NS = 1, info.num_subcores
    NW = NC * NS
    assert B % NW == 0
    b_per_w = B // NW
    nch = _NCHUNK
    rc = b_per_w // nch
    assert rc * nch == b_per_w and rc % 8 == 0
    mesh = plsc.VectorSubcoreMesh(
        core_axis_name="c", subcore_axis_name="s", num_cores=1
    )

    @functools.partial(
        pl.kernel,
        mesh=mesh,
        out_type=jax.ShapeDtypeStruct((B, D), jnp.float32),
        scratch_types=[
            pltpu.VMEM((b_per_w,), jnp.int32),
            pltpu.VMEM((b_per_w, D), jnp.float32),
        ]
        + [pltpu.SemaphoreType.DMA] * (2 * nch),
    )
    def k(table_hbm, idx_hbm, out_hbm, idx_v, rows_v, *sems):
        gsem, osem = sems[:nch], sems[nch:]
        wid = lax.axis_index("s") * NC + lax.axis_index("c")
        base = wid * b_per_w
        pltpu.sync_copy(idx_hbm.at[pl.ds(base, b_per_w)], idx_v)
        gathers = [
            pltpu.async_copy(
                table_hbm.at[idx_v.at[pl.ds(c * rc, rc)]],
                rows_v.at[pl.ds(c * rc, rc)],
                gsem[c],
            )
            for c in range(nch)
        ]
        outs = []
        for c in range(nch):
            gathers[c].wait()
            outs.append(
                pltpu.async_copy(
                    rows_v.at[pl.ds(c * rc, rc)],
                    out_hbm.at[pl.ds(base + c * rc, rc)],
                    osem[c],
                )
            )
        for o in outs:
            o.wait()

    return k


def kernel(labels, table):
    V, D = table.shape
    (B,) = labels.shape
    k = _make_gather(V, D, B)
    return k(table, labels.astype(jnp.int32))
